# Initial kernel scaffold; baseline (speedup 1.0000x reference)
import jax
import jax.numpy as jnp
from jax.experimental import pallas as pl


def kernel(logits, temperature, top_p, top_k):
    raise NotImplementedError("write your pallas kernel here")


# speedup vs baseline: 89.7400x; 89.7400x over previous
import functools

import jax
import jax.numpy as jnp
from jax import lax
from jax.experimental import pallas as pl
from jax.experimental.pallas import tpu as pltpu
from jax.experimental.pallas import tpu_sc as plsc

_ROWS = 128
_VOCAB = 100000
_LANES = 16
_NC = 2
_NS = 16
_NW = _NC * _NS
_ROWS_PER_W = _ROWS // _NW
_VECS = _VOCAB // _LANES
_IMAX = 2**31 - 1


def _make_sc_argmax():
    mesh = plsc.VectorSubcoreMesh(core_axis_name="c", subcore_axis_name="s")

    @functools.partial(
        pl.kernel,
        mesh=mesh,
        out_type=jax.ShapeDtypeStruct((_NW, _LANES), jnp.int32),
        compiler_params=pltpu.CompilerParams(needs_layout_passes=False),
        scratch_types=[
            pltpu.VMEM((_VOCAB,), jnp.float32),
            pltpu.VMEM((_LANES,), jnp.int32),
        ],
    )
    def body(logits_hbm, out_hbm, buf, res_ref):
        wid = lax.axis_index("s") * _NC + lax.axis_index("c")
        lanes = lax.iota(jnp.int32, _LANES)
        res_ref[...] = jnp.zeros((_LANES,), jnp.int32)

        for j in range(_ROWS_PER_W):
            row = wid * _ROWS_PER_W + j
            pltpu.sync_copy(logits_hbm.at[row], buf)

            def step(i, carry):
                m, bi = carry
                v = buf[pl.ds(i * _LANES, _LANES)]
                gt = v > m
                m = jnp.where(gt, v, m)
                bi = jnp.where(gt, lanes + i * _LANES, bi)
                return (m, bi)

            m0 = jnp.full((_LANES,), -jnp.inf, jnp.float32)
            b0 = jnp.zeros((_LANES,), jnp.int32)
            m, bi = lax.fori_loop(0, _VECS, step, (m0, b0))

            k_sorted, _ = plsc.sort_key_val(m, bi, descending=True)
            rowmax = k_sorted[0]
            cand = jnp.where(m == rowmax, bi, _IMAX)
            c_sorted, _ = plsc.sort_key_val(cand, cand)
            rowidx = c_sorted[0]
            res_ref[...] = jnp.where(lanes == j, rowidx, res_ref[...])

        pltpu.sync_copy(res_ref, out_hbm.at[wid])

    return body


_SC_ARGMAX = _make_sc_argmax()


def kernel(logits, temperature, top_p, top_k):
    del temperature, top_p, top_k
    out = _SC_ARGMAX(logits)
    return out[:, :_ROWS_PER_W].reshape(_ROWS, 1)


# speedup vs baseline: 107.5870x; 1.1989x over previous
import functools

import jax
import jax.numpy as jnp
from jax import lax
from jax.experimental import pallas as pl
from jax.experimental.pallas import tpu as pltpu
from jax.experimental.pallas import tpu_sc as plsc

_ROWS = 128
_VOCAB = 100000
_LANES = 16
_NC = 2
_NS = 16
_NW = _NC * _NS
_ROWS_PER_W = _ROWS // _NW
_CHUNK = 50000
_CPR = _VOCAB // _CHUNK
_CVECS = _CHUNK // _LANES
_IMAX = 2**31 - 1
_UNROLL = 25


def _make_sc_argmax():
    mesh = plsc.VectorSubcoreMesh(core_axis_name="c", subcore_axis_name="s")

    @functools.partial(
        pl.kernel,
        mesh=mesh,
        out_type=jax.ShapeDtypeStruct((_NW, _LANES), jnp.int32),
        compiler_params=pltpu.CompilerParams(needs_layout_passes=False),
        scratch_types=[
            pltpu.VMEM((_CHUNK,), jnp.float32),
            pltpu.VMEM((_CHUNK,), jnp.float32),
            pltpu.VMEM((_LANES,), jnp.int32),
            pltpu.SemaphoreType.DMA,
            pltpu.SemaphoreType.DMA,
        ],
    )
    def body(logits_hbm, out_hbm, buf0, buf1, res_ref, sem0, sem1):
        wid = lax.axis_index("s") * _NC + lax.axis_index("c")
        lanes = lax.iota(jnp.int32, _LANES)
        res_ref[...] = jnp.zeros((_LANES,), jnp.int32)

        bufs = (buf0, buf1)
        sems = (sem0, sem1)
        nt = _ROWS_PER_W * _CPR

        def src(t):
            row = wid * _ROWS_PER_W + t // _CPR
            base = pl.multiple_of(row * _VOCAB + (t % _CPR) * _CHUNK, 8)
            return logits_hbm.at[pl.ds(base, _CHUNK)]

        def scan_chunk(buf, state):
            def step(i, carry):
                m, bi, idx = carry
                v = buf[pl.ds(i * _LANES, _LANES)]
                gt = v > m
                m = jnp.where(gt, v, m)
                bi = jnp.where(gt, idx, bi)
                return (m, bi, idx + _LANES)

            return lax.fori_loop(0, _CVECS, step, state, unroll=_UNROLL)

        handles = [None] * nt
        handles[0] = pltpu.async_copy(src(0), bufs[0], sems[0])
        for r in range(_ROWS_PER_W):
            state = (
                jnp.full((_LANES,), -jnp.inf, jnp.float32),
                jnp.zeros((_LANES,), jnp.int32),
                lanes,
            )
            for c in range(_CPR):
                t = r * _CPR + c
                if t + 1 < nt:
                    handles[t + 1] = pltpu.async_copy(
                        src(t + 1), bufs[(t + 1) % 2], sems[(t + 1) % 2])
                handles[t].wait()
                state = scan_chunk(bufs[t % 2], state)

            m, bi, _ = state
            k_sorted, _ = plsc.sort_key_val(m, bi, descending=True)
            rowmax = k_sorted[0]
            cand = jnp.where(m == rowmax, bi, _IMAX)
            c_sorted, _ = plsc.sort_key_val(cand, cand)
            rowidx = c_sorted[0]
            res_ref[...] = jnp.where(lanes == r, rowidx, res_ref[...])

        pltpu.sync_copy(res_ref, out_hbm.at[wid])

    return body


_SC_ARGMAX = _make_sc_argmax()


def kernel(logits, temperature, top_p, top_k):
    del temperature, top_p, top_k
    out = _SC_ARGMAX(logits.reshape(_ROWS * _VOCAB))
    return out[:, :_ROWS_PER_W].reshape(_ROWS, 1)


# speedup vs baseline: 175.1672x; 1.6281x over previous
import functools

import jax
import jax.numpy as jnp
from jax import lax
from jax.experimental import pallas as pl
from jax.experimental.pallas import tpu as pltpu
from jax.experimental.pallas import tpu_sc as plsc

_ROWS = 128
_VOCAB = 100000
_LANES = 16
_NC = 2
_NS = 16
_NW = _NC * _NS
_ROWS_PER_W = _ROWS // _NW

_CSIZE = 51200
_CHUNKS = ((0, _CSIZE), (99840 - _CSIZE, _CSIZE))
_CPR = len(_CHUNKS)
_TAIL0 = 99840
_TAILN = 256
_IMAX = 2**31 - 1
_UNROLL = 25


def _make_sc_argmax():
    mesh = plsc.VectorSubcoreMesh(core_axis_name="c", subcore_axis_name="s")

    @functools.partial(
        pl.kernel,
        mesh=mesh,
        out_type=jax.ShapeDtypeStruct((_NW, _LANES), jnp.int32),
        compiler_params=pltpu.CompilerParams(needs_layout_passes=False),
        scratch_types=[
            pltpu.VMEM((_CSIZE,), jnp.float32),
            pltpu.VMEM((_CSIZE,), jnp.float32),
            pltpu.VMEM((_ROWS_PER_W * _TAILN,), jnp.float32),
            pltpu.VMEM((_LANES,), jnp.int32),
            pltpu.SemaphoreType.DMA,
            pltpu.SemaphoreType.DMA,
            pltpu.SemaphoreType.DMA,
        ],
    )
    def body(logits_hbm, tail_hbm, out_hbm, buf0, buf1, tailbuf, res_ref,
             sem0, sem1, sem2):
        wid = lax.axis_index("s") * _NC + lax.axis_index("c")
        lanes = lax.iota(jnp.int32, _LANES)
        res_ref[...] = jnp.zeros((_LANES,), jnp.int32)

        bufs = (buf0, buf1)
        sems = (sem0, sem1)
        nt = _ROWS_PER_W * _CPR

        def start(t):
            row = wid * _ROWS_PER_W + t // _CPR
            off, size = _CHUNKS[t % _CPR]
            return pltpu.async_copy(
                logits_hbm.at[row].at[pl.ds(off, size)],
                bufs[t % 2], sems[t % 2])

        def scan(buf, buf_off, nvec, m, bi, idx0, unroll):
            def step(i, carry):
                m, bi, idx = carry
                v = buf[pl.ds(buf_off + i * _LANES, _LANES)]
                gt = v > m
                m = jnp.where(gt, v, m)
                bi = jnp.where(gt, idx, bi)
                return (m, bi, idx + _LANES)

            m, bi, _ = lax.fori_loop(0, nvec, step, (m, bi, lanes + idx0),
                                     unroll=unroll)
            return m, bi

        tail_handles = []
        for r in range(_ROWS_PER_W):
            row = wid * _ROWS_PER_W + r
            tail_handles.append(pltpu.async_copy(
                tail_hbm.at[row], tailbuf.at[pl.ds(r * _TAILN, _TAILN)],
                sem2))

        handles = [None] * nt
        handles[0] = start(0)
        for r in range(_ROWS_PER_W):
            m = jnp.full((_LANES,), -jnp.inf, jnp.float32)
            bi = jnp.zeros((_LANES,), jnp.int32)
            for c in range(_CPR):
                t = r * _CPR + c
                if t + 1 < nt:
                    handles[t + 1] = start(t + 1)
                handles[t].wait()
                m, bi = scan(bufs[t % 2], 0, _CSIZE // _LANES, m, bi,
                             _CHUNKS[c][0], _UNROLL)

            tail_handles[r].wait()
            m, bi = scan(tailbuf, r * _TAILN, _TAILN // _LANES, m, bi,
                         _TAIL0, _TAILN // _LANES)

            k_sorted, _ = plsc.sort_key_val(m, bi, descending=True)
            rowmax = k_sorted[0]
            cand = jnp.where(m == rowmax, bi, _IMAX)
            c_sorted, _ = plsc.sort_key_val(cand, cand)
            rowidx = c_sorted[0]
            res_ref[...] = jnp.where(lanes == r, rowidx, res_ref[...])

        pltpu.sync_copy(res_ref, out_hbm.at[wid])

    return body


_SC_ARGMAX = _make_sc_argmax()


def kernel(logits, temperature, top_p, top_k):
    del temperature, top_p, top_k
    tail = lax.slice(logits, (0, _TAIL0), (_ROWS, _VOCAB))
    tail = jnp.concatenate(
        [tail, jnp.full((_ROWS, _TAILN - (_VOCAB - _TAIL0)), -jnp.inf,
                        jnp.float32)], axis=1)
    out = _SC_ARGMAX(logits, tail)
    return out[:, :_ROWS_PER_W].reshape(_ROWS, 1)
